# baseline (speedup 1.0000x reference)
import functools

import jax
import jax.numpy as jnp
from jax import lax
from jax.experimental import pallas as pl
from jax.experimental.pallas import tpu as pltpu
from jax.experimental.pallas import tpu_sc as plsc

EPAD = 128
BM = 256
NBLK = 128
NW = 32


def _gate_body(xf_ref, gwt_ref, w2f_ref,
               pos0_ref, pos1_ref, wts_ref, blk_ref, w2b_ref, st_ref,
               *, n_experts):
    w2b_ref[...] = w2f_ref[...].astype(jnp.bfloat16)
    p = pl.program_id(0)
    i = pl.program_id(1)
    nt = pl.num_programs(1)
    tb = xf_ref.shape[0]

    x32 = xf_ref[...]
    logits = jax.lax.dot_general(
        x32, gwt_ref[...], (((1,), (0,)), ((), ())),
        preferred_element_type=jnp.float32)
    lane = jax.lax.broadcasted_iota(jnp.int32, logits.shape, 1)
    valid = lane < n_experts
    l = jnp.where(valid, logits, -1e30)
    m = jnp.max(l, axis=1, keepdims=True)
    ex = jnp.where(valid, jnp.exp(l - m), 0.0)
    sc = ex / jnp.sum(ex, axis=1, keepdims=True)
    sc = jnp.where(valid, sc, -1.0)
    i1 = jnp.argmax(sc, axis=1)[:, None]
    oh1f = (lane == i1).astype(jnp.float32)
    i2 = jnp.argmax(jnp.where(lane == i1, -2.0, sc), axis=1)[:, None]
    oh2f = (lane == i2).astype(jnp.float32)
    ohsum = oh1f + oh2f
    colsum = jnp.sum(ohsum, axis=0, keepdims=True)

    @pl.when(p == 0)
    def _count_pass():
        prev = jnp.where(i == 0, 0.0, st_ref[0:1, :])
        counts = prev + colsum
        st_ref[0:1, :] = counts

        @pl.when(i == nt - 1)
        def _finalize():
            ca = jnp.floor((counts + float(BM - 1)) * (1.0 / BM)) * BM
            r128i = jax.lax.broadcasted_iota(jnp.int32, (EPAD, EPAD), 0)
            c128i = jax.lax.broadcasted_iota(jnp.int32, (EPAD, EPAD), 1)
            upper = (r128i < c128i).astype(jnp.float32)
            starts = jax.lax.dot_general(
                ca, upper, (((1,), (0,)), ((), ())),
                preferred_element_type=jnp.float32)
            st_ref[1:2, :] = starts
            ends_b = (starts + ca) * (1.0 / BM)
            lane1 = jax.lax.broadcasted_iota(jnp.int32, ends_b.shape, 1)
            ends_b = jnp.where(lane1 < n_experts, ends_b, 1e9)
            cmp = (r128i.astype(jnp.float32)
                   >= jnp.broadcast_to(ends_b, (EPAD, EPAD)))
            blk = jnp.sum(cmp.astype(jnp.float32), axis=1, keepdims=True)
            blk_ref[...] = jnp.minimum(blk, n_experts - 1).astype(jnp.int32)

    @pl.when(p == 1)
    def _emit_pass():
        carry = jnp.where(i == 0, 0.0, st_ref[2:3, :])
        rt = jax.lax.broadcasted_iota(jnp.int32, (tb, tb), 0)
        ct = jax.lax.broadcasted_iota(jnp.int32, (tb, tb), 1)
        ltri = (rt > ct).astype(jnp.float32)
        cum = jax.lax.dot_general(
            ltri, ohsum, (((1,), (0,)), ((), ())),
            preferred_element_type=jnp.float32)
        tot = cum + carry + st_ref[1:2, :]
        pos0_ref[...] = jnp.sum(oh1f * tot, axis=1,
                                keepdims=True).astype(jnp.int32)
        pos1_ref[...] = jnp.sum(oh2f * tot, axis=1,
                                keepdims=True).astype(jnp.int32)
        w1 = jnp.sum(oh1f * sc, axis=1, keepdims=True)
        w2 = jnp.sum(oh2f * sc, axis=1, keepdims=True)
        wts_ref[...] = jnp.concatenate([w1, w2], axis=1)
        st_ref[2:3, :] = carry + colsum


def _sc_scatter(x2, pos0, pos1, npad):
    t, d = x2.shape
    bw = t // NW
    hw = bw // 2
    mesh = plsc.VectorSubcoreMesh(core_axis_name="c", subcore_axis_name="s")

    @functools.partial(
        pl.kernel, mesh=mesh,
        out_type=jax.ShapeDtypeStruct((npad, d), jnp.float32),
        scratch_types=[
            pltpu.VMEM((hw,), jnp.int32),
            pltpu.VMEM((hw,), jnp.int32),
            pltpu.VMEM((hw, d), jnp.float32),
            pltpu.SemaphoreType.DMA,
        ],
    )
    def k(x_hbm, p0_hbm, p1_hbm, xs_hbm, i0_v, i1_v, rows_v, sem):
        wid = lax.axis_index("s") * 2 + lax.axis_index("c")
        for h in range(2):
            base = wid * bw + h * hw
            pltpu.sync_copy(p0_hbm.at[pl.ds(base, hw)], i0_v)
            pltpu.sync_copy(p1_hbm.at[pl.ds(base, hw)], i1_v)
            pltpu.sync_copy(x_hbm.at[pl.ds(base, hw)], rows_v)
            pltpu.async_copy(rows_v, xs_hbm.at[i0_v], sem).wait()
            pltpu.async_copy(rows_v, xs_hbm.at[i1_v], sem).wait()

    return k(x2, pos0, pos1)


def _sc_gather2(eo2, pos0, pos1, t):
    d = eo2.shape[1]
    bw = t // NW
    hw = bw // 2
    mesh = plsc.VectorSubcoreMesh(core_axis_name="c", subcore_axis_name="s")
    row_t = jax.ShapeDtypeStruct((t, d), jnp.float32)

    @functools.partial(
        pl.kernel, mesh=mesh,
        out_type=(row_t, row_t),
        scratch_types=[
            pltpu.VMEM((hw,), jnp.int32),
            pltpu.VMEM((hw, d), jnp.float32),
            pltpu.SemaphoreType.DMA,
        ],
    )
    def k(eo_hbm, p0_hbm, p1_hbm, ag_hbm, bg_hbm, idx_v, rows_v, sem):
        wid = lax.axis_index("s") * 2 + lax.axis_index("c")
        for h in range(2):
            base = wid * bw + h * hw
            pltpu.sync_copy(p0_hbm.at[pl.ds(base, hw)], idx_v)
            pltpu.async_copy(eo_hbm.at[idx_v], rows_v, sem).wait()
            pltpu.sync_copy(rows_v, ag_hbm.at[pl.ds(base, hw)])
            pltpu.sync_copy(p1_hbm.at[pl.ds(base, hw)], idx_v)
            pltpu.async_copy(eo_hbm.at[idx_v], rows_v, sem).wait()
            pltpu.sync_copy(rows_v, bg_hbm.at[pl.ds(base, hw)])

    return k(eo2, pos0, pos1)


def _shared_body(xf_ref, ws1_ref, ws2_ref, bs1_ref, bs2_ref, z_ref):
    xb = xf_ref[...].astype(jnp.bfloat16)
    ws1 = ws1_ref[...].astype(jnp.bfloat16)
    ws2 = ws2_ref[...].astype(jnp.bfloat16)
    h1 = jax.lax.dot_general(xb, ws1, (((1,), (1,)), ((), ())),
                             preferred_element_type=jnp.float32) + bs1_ref[...]
    h = (h1 * jax.nn.sigmoid(h1)).astype(jnp.bfloat16)
    z = jax.lax.dot_general(h, ws2, (((1,), (1,)), ((), ())),
                            preferred_element_type=jnp.float32) + bs2_ref[...]
    z_ref[...] = z.astype(jnp.bfloat16)


def _grouped_body(blk_ref, xs_ref, w1_ref, w3_ref, w2_ref,
                  b1_ref, b3_ref, b2_ref, eo_ref,
                  w1s_ref, w3s_ref):
    i = pl.program_id(0)
    changed = jnp.logical_or(
        i == 0, blk_ref[i] != blk_ref[jnp.maximum(i - 1, 0)])

    @pl.when(changed)
    def _cast():
        w1s_ref[...] = w1_ref[0].astype(jnp.bfloat16)
        w3s_ref[...] = w3_ref[0].astype(jnp.bfloat16)

    xb = xs_ref[...].astype(jnp.bfloat16)
    h1 = jax.lax.dot_general(xb, w1s_ref[...], (((1,), (1,)), ((), ())),
                             preferred_element_type=jnp.float32) + b1_ref[0]
    h3 = jax.lax.dot_general(xb, w3s_ref[...], (((1,), (1,)), ((), ())),
                             preferred_element_type=jnp.float32) + b3_ref[0]
    h = (h1 * jax.nn.sigmoid(h1) * h3).astype(jnp.bfloat16)
    eo = jax.lax.dot_general(h, w2_ref[0], (((1,), (1,)), ((), ())),
                             preferred_element_type=jnp.float32) + b2_ref[0]
    eo_ref[...] = eo


def _combine_body(ag_ref, bg_ref, z_ref, wts_ref, y_ref):
    w = wts_ref[...]
    y_ref[...] = (w[:, 0:1] * ag_ref[...] + w[:, 1:2] * bg_ref[...]
                  + z_ref[...].astype(jnp.float32))


def kernel(embeddings, x, gate_w, W1, b1, W2, b2, W3, b3, Ws1, bs1, Ws2, bs2):
    del embeddings
    shape = x.shape
    dim = shape[-1]
    xf = x.reshape(-1, dim)
    t = xf.shape[0]
    n_experts, inter = W1.shape[0], W1.shape[1]
    npad = 2 * t + n_experts * BM
    nb = npad // BM

    tb = t // n_experts
    hi = inter // 2
    gwt = jnp.zeros((dim, EPAD), jnp.float32).at[:, :n_experts].set(gate_w.T)

    hd = dim // 2
    pos0, pos1, wts, blk, w2b = pl.pallas_call(
        functools.partial(_gate_body, n_experts=n_experts),
        grid=(2, t // tb),
        in_specs=[
            pl.BlockSpec((tb, dim), lambda p, i: (i, 0)),
            pl.BlockSpec((dim, EPAD), lambda p, i: (0, 0)),
            pl.BlockSpec((1, hd, inter), lambda p, i: (i, p, 0)),
        ],
        out_specs=[
            pl.BlockSpec((tb, 1), lambda p, i: (i, 0)),
            pl.BlockSpec((tb, 1), lambda p, i: (i, 0)),
            pl.BlockSpec((tb, 2), lambda p, i: (i, 0)),
            pl.BlockSpec((NBLK, 1), lambda p, i: (0, 0)),
            pl.BlockSpec((1, hd, inter), lambda p, i: (i, p, 0)),
        ],
        out_shape=[
            jax.ShapeDtypeStruct((t, 1), jnp.int32),
            jax.ShapeDtypeStruct((t, 1), jnp.int32),
            jax.ShapeDtypeStruct((t, 2), jnp.float32),
            jax.ShapeDtypeStruct((NBLK, 1), jnp.int32),
            jax.ShapeDtypeStruct((n_experts, dim, inter), jnp.bfloat16),
        ],
        scratch_shapes=[pltpu.VMEM((8, EPAD), jnp.float32)],
        compiler_params=pltpu.CompilerParams(
            dimension_semantics=("arbitrary", "arbitrary")),
    )(xf, gwt, W2)

    pos0 = pos0.reshape(t)
    pos1 = pos1.reshape(t)
    th = t // 2

    xs = _sc_scatter(xf, pos0, pos1, npad)

    def shared_half(half):
        return pl.pallas_call(
            _shared_body,
            grid=(th // tb,),
            in_specs=[
                pl.BlockSpec((tb, dim),
                             lambda i, h=half: (i + h * (th // tb), 0)),
                pl.BlockSpec((inter, dim), lambda i: (0, 0)),
                pl.BlockSpec((dim, inter), lambda i: (0, 0)),
                pl.BlockSpec((1, inter), lambda i: (0, 0)),
                pl.BlockSpec((1, dim), lambda i: (0, 0)),
            ],
            out_specs=pl.BlockSpec((tb, dim), lambda i: (i, 0)),
            out_shape=jax.ShapeDtypeStruct((th, dim), jnp.bfloat16),
        )(xf, Ws1, Ws2, bs1.reshape(1, inter), bs2.reshape(1, dim))

    z0 = shared_half(0)

    eo = pl.pallas_call(
        _grouped_body,
        grid_spec=pltpu.PrefetchScalarGridSpec(
            num_scalar_prefetch=1,
            grid=(nb,),
            in_specs=[
                pl.BlockSpec((BM, dim), lambda i, be: (i, 0)),
                pl.BlockSpec((1, inter, dim), lambda i, be: (be[i], 0, 0)),
                pl.BlockSpec((1, inter, dim), lambda i, be: (be[i], 0, 0)),
                pl.BlockSpec((1, dim, inter), lambda i, be: (be[i], 0, 0)),
                pl.BlockSpec((1, 1, inter), lambda i, be: (be[i], 0, 0)),
                pl.BlockSpec((1, 1, inter), lambda i, be: (be[i], 0, 0)),
                pl.BlockSpec((1, 1, dim), lambda i, be: (be[i], 0, 0)),
            ],
            out_specs=pl.BlockSpec((BM, dim), lambda i, be: (i, 0)),
            scratch_shapes=[
                pltpu.VMEM((inter, dim), jnp.bfloat16),
                pltpu.VMEM((inter, dim), jnp.bfloat16),
            ],
        ),
        out_shape=jax.ShapeDtypeStruct((npad, dim), jnp.float32),
        compiler_params=pltpu.CompilerParams(
            dimension_semantics=("arbitrary",)),
    )(blk.reshape(-1)[:nb], xs, W1, W3, w2b,
      b1.reshape(n_experts, 1, inter), b3.reshape(n_experts, 1, inter),
      b2.reshape(n_experts, 1, dim))

    ag0, bg0 = _sc_gather2(eo, pos0[:th], pos1[:th], th)
    z1 = shared_half(1)
    ag1, bg1 = _sc_gather2(eo, pos0[th:], pos1[th:], th)

    def combine_half(agh, bgh, zh, half):
        return pl.pallas_call(
            _combine_body,
            grid=(th // tb,),
            in_specs=[
                pl.BlockSpec((tb, dim), lambda i: (i, 0)),
                pl.BlockSpec((tb, dim), lambda i: (i, 0)),
                pl.BlockSpec((tb, dim), lambda i: (i, 0)),
                pl.BlockSpec((tb, 2),
                             lambda i, h=half: (i + h * (th // tb), 0)),
            ],
            out_specs=pl.BlockSpec((tb, dim), lambda i: (i, 0)),
            out_shape=jax.ShapeDtypeStruct((th, dim), jnp.float32),
        )(agh, bgh, zh, wts)

    y0 = combine_half(ag0, bg0, z0, 0)
    y1 = combine_half(ag1, bg1, z1, 1)

    return jnp.concatenate([y0, y1], axis=0).reshape(shape)


# speedup vs baseline: 1.0966x; 1.0966x over previous
import functools

import jax
import jax.numpy as jnp
from jax import lax
from jax.experimental import pallas as pl
from jax.experimental.pallas import tpu as pltpu
from jax.experimental.pallas import tpu_sc as plsc

EPAD = 128
BM = 256
NBLK = 128
NW = 32


def _gate_body(xf_ref, gwt_ref, pos0_ref, pos1_ref, wts_ref, blk_ref, st_ref,
               *, n_experts):
    p = pl.program_id(0)
    i = pl.program_id(1)
    nt = pl.num_programs(1)
    tb = xf_ref.shape[0]

    x32 = xf_ref[...]
    logits = jax.lax.dot_general(
        x32, gwt_ref[...], (((1,), (0,)), ((), ())),
        preferred_element_type=jnp.float32)
    lane = jax.lax.broadcasted_iota(jnp.int32, logits.shape, 1)
    valid = lane < n_experts
    l = jnp.where(valid, logits, -1e30)
    m = jnp.max(l, axis=1, keepdims=True)
    ex = jnp.where(valid, jnp.exp(l - m), 0.0)
    sc = ex / jnp.sum(ex, axis=1, keepdims=True)
    sc = jnp.where(valid, sc, -1.0)
    i1 = jnp.argmax(sc, axis=1)[:, None]
    oh1f = (lane == i1).astype(jnp.float32)
    i2 = jnp.argmax(jnp.where(lane == i1, -2.0, sc), axis=1)[:, None]
    oh2f = (lane == i2).astype(jnp.float32)
    ohsum = oh1f + oh2f
    colsum = jnp.sum(ohsum, axis=0, keepdims=True)

    @pl.when(p == 0)
    def _count_pass():
        prev = jnp.where(i == 0, 0.0, st_ref[0:1, :])
        counts = prev + colsum
        st_ref[0:1, :] = counts

        @pl.when(i == nt - 1)
        def _finalize():
            ca = jnp.floor((counts + float(BM - 1)) * (1.0 / BM)) * BM
            r128i = jax.lax.broadcasted_iota(jnp.int32, (EPAD, EPAD), 0)
            c128i = jax.lax.broadcasted_iota(jnp.int32, (EPAD, EPAD), 1)
            upper = (r128i < c128i).astype(jnp.float32)
            starts = jax.lax.dot_general(
                ca, upper, (((1,), (0,)), ((), ())),
                preferred_element_type=jnp.float32)
            st_ref[1:2, :] = starts
            ends_b = (starts + ca) * (1.0 / BM)
            lane1 = jax.lax.broadcasted_iota(jnp.int32, ends_b.shape, 1)
            ends_b = jnp.where(lane1 < n_experts, ends_b, 1e9)
            cmp = (r128i.astype(jnp.float32)
                   >= jnp.broadcast_to(ends_b, (EPAD, EPAD)))
            blk = jnp.sum(cmp.astype(jnp.float32), axis=1, keepdims=True)
            blk_ref[...] = jnp.minimum(blk, n_experts - 1).astype(jnp.int32)

    @pl.when(p == 1)
    def _emit_pass():
        carry = jnp.where(i == 0, 0.0, st_ref[2:3, :])
        rt = jax.lax.broadcasted_iota(jnp.int32, (tb, tb), 0)
        ct = jax.lax.broadcasted_iota(jnp.int32, (tb, tb), 1)
        ltri = (rt > ct).astype(jnp.float32)
        cum = jax.lax.dot_general(
            ltri, ohsum, (((1,), (0,)), ((), ())),
            preferred_element_type=jnp.float32)
        tot = cum + carry + st_ref[1:2, :]
        pos0_ref[...] = jnp.sum(oh1f * tot, axis=1,
                                keepdims=True).astype(jnp.int32)
        pos1_ref[...] = jnp.sum(oh2f * tot, axis=1,
                                keepdims=True).astype(jnp.int32)
        w1 = jnp.sum(oh1f * sc, axis=1, keepdims=True)
        w2 = jnp.sum(oh2f * sc, axis=1, keepdims=True)
        wts_ref[...] = jnp.concatenate([w1, w2], axis=1)
        st_ref[2:3, :] = carry + colsum


def _sc_scatter(x2, pos0, pos1, npad):
    t, d = x2.shape
    bw = t // NW
    hw = bw // 2
    mesh = plsc.VectorSubcoreMesh(core_axis_name="c", subcore_axis_name="s")

    @functools.partial(
        pl.kernel, mesh=mesh,
        out_type=jax.ShapeDtypeStruct((npad, d), jnp.float32),
        scratch_types=[
            pltpu.VMEM((hw,), jnp.int32),
            pltpu.VMEM((hw,), jnp.int32),
            pltpu.VMEM((hw, d), jnp.float32),
            pltpu.SemaphoreType.DMA,
        ],
    )
    def k(x_hbm, p0_hbm, p1_hbm, xs_hbm, i0_v, i1_v, rows_v, sem):
        wid = lax.axis_index("s") * 2 + lax.axis_index("c")
        for h in range(2):
            base = wid * bw + h * hw
            pltpu.sync_copy(p0_hbm.at[pl.ds(base, hw)], i0_v)
            pltpu.sync_copy(p1_hbm.at[pl.ds(base, hw)], i1_v)
            pltpu.sync_copy(x_hbm.at[pl.ds(base, hw)], rows_v)
            pltpu.async_copy(rows_v, xs_hbm.at[i0_v], sem).wait()
            pltpu.async_copy(rows_v, xs_hbm.at[i1_v], sem).wait()

    return k(x2, pos0, pos1)


def _sc_gather2(eo2, pos0, pos1, t):
    d = eo2.shape[1]
    bw = t // NW
    hw = bw // 2
    mesh = plsc.VectorSubcoreMesh(core_axis_name="c", subcore_axis_name="s")
    row_t = jax.ShapeDtypeStruct((t, d), jnp.float32)

    @functools.partial(
        pl.kernel, mesh=mesh,
        out_type=(row_t, row_t),
        scratch_types=[
            pltpu.VMEM((hw,), jnp.int32),
            pltpu.VMEM((hw, d), jnp.float32),
            pltpu.SemaphoreType.DMA,
        ],
    )
    def k(eo_hbm, p0_hbm, p1_hbm, ag_hbm, bg_hbm, idx_v, rows_v, sem):
        wid = lax.axis_index("s") * 2 + lax.axis_index("c")
        for h in range(2):
            base = wid * bw + h * hw
            pltpu.sync_copy(p0_hbm.at[pl.ds(base, hw)], idx_v)
            pltpu.async_copy(eo_hbm.at[idx_v], rows_v, sem).wait()
            pltpu.sync_copy(rows_v, ag_hbm.at[pl.ds(base, hw)])
            pltpu.sync_copy(p1_hbm.at[pl.ds(base, hw)], idx_v)
            pltpu.async_copy(eo_hbm.at[idx_v], rows_v, sem).wait()
            pltpu.sync_copy(rows_v, bg_hbm.at[pl.ds(base, hw)])

    return k(eo2, pos0, pos1)


def _shared_body(xf_ref, ws1_ref, ws2_ref, bs1_ref, bs2_ref, z_ref):
    xb = xf_ref[...].astype(jnp.bfloat16)
    ws1 = ws1_ref[...].astype(jnp.bfloat16)
    ws2 = ws2_ref[...].astype(jnp.bfloat16)
    h1 = jax.lax.dot_general(xb, ws1, (((1,), (1,)), ((), ())),
                             preferred_element_type=jnp.float32) + bs1_ref[...]
    h = (h1 * jax.nn.sigmoid(h1)).astype(jnp.bfloat16)
    z = jax.lax.dot_general(h, ws2, (((1,), (1,)), ((), ())),
                            preferred_element_type=jnp.float32) + bs2_ref[...]
    z_ref[...] = z.astype(jnp.bfloat16)


def _grouped_body(blk_ref, xs_ref, w1_ref, w3_ref, w2_ref,
                  b1_ref, b3_ref, b2_ref, eo_ref,
                  w1s_ref, w3s_ref, w2s_ref):
    i = pl.program_id(0)
    changed = jnp.logical_or(
        i == 0, blk_ref[i] != blk_ref[jnp.maximum(i - 1, 0)])

    @pl.when(changed)
    def _cast():
        w1s_ref[...] = w1_ref[0].astype(jnp.bfloat16)
        w3s_ref[...] = w3_ref[0].astype(jnp.bfloat16)
        w2s_ref[...] = w2_ref[0].astype(jnp.bfloat16)

    xb = xs_ref[...].astype(jnp.bfloat16)
    h1 = jax.lax.dot_general(xb, w1s_ref[...], (((1,), (1,)), ((), ())),
                             preferred_element_type=jnp.float32) + b1_ref[0]
    h3 = jax.lax.dot_general(xb, w3s_ref[...], (((1,), (1,)), ((), ())),
                             preferred_element_type=jnp.float32) + b3_ref[0]
    h = (h1 * jax.nn.sigmoid(h1) * h3).astype(jnp.bfloat16)
    eo = jax.lax.dot_general(h, w2s_ref[...], (((1,), (1,)), ((), ())),
                             preferred_element_type=jnp.float32) + b2_ref[0]
    eo_ref[...] = eo


def _combine_body(ag_ref, bg_ref, z_ref, wts_ref, y_ref):
    w = wts_ref[...]
    y_ref[...] = (w[:, 0:1] * ag_ref[...] + w[:, 1:2] * bg_ref[...]
                  + z_ref[...].astype(jnp.float32))


def kernel(embeddings, x, gate_w, W1, b1, W2, b2, W3, b3, Ws1, bs1, Ws2, bs2):
    del embeddings
    shape = x.shape
    dim = shape[-1]
    xf = x.reshape(-1, dim)
    t = xf.shape[0]
    n_experts, inter = W1.shape[0], W1.shape[1]
    npad = 2 * t + n_experts * BM
    nb = npad // BM

    tb = t // n_experts
    hi = inter // 2
    gwt = jnp.zeros((dim, EPAD), jnp.float32).at[:, :n_experts].set(gate_w.T)

    pos0, pos1, wts, blk = pl.pallas_call(
        functools.partial(_gate_body, n_experts=n_experts),
        grid=(2, t // tb),
        in_specs=[
            pl.BlockSpec((tb, dim), lambda p, i: (i, 0)),
            pl.BlockSpec((dim, EPAD), lambda p, i: (0, 0)),
        ],
        out_specs=[
            pl.BlockSpec((tb, 1), lambda p, i: (i, 0)),
            pl.BlockSpec((tb, 1), lambda p, i: (i, 0)),
            pl.BlockSpec((tb, 2), lambda p, i: (i, 0)),
            pl.BlockSpec((NBLK, 1), lambda p, i: (0, 0)),
        ],
        out_shape=[
            jax.ShapeDtypeStruct((t, 1), jnp.int32),
            jax.ShapeDtypeStruct((t, 1), jnp.int32),
            jax.ShapeDtypeStruct((t, 2), jnp.float32),
            jax.ShapeDtypeStruct((NBLK, 1), jnp.int32),
        ],
        scratch_shapes=[pltpu.VMEM((8, EPAD), jnp.float32)],
        compiler_params=pltpu.CompilerParams(
            dimension_semantics=("arbitrary", "arbitrary")),
    )(xf, gwt)

    pos0 = pos0.reshape(t)
    pos1 = pos1.reshape(t)

    xs = _sc_scatter(xf, pos0, pos1, npad)

    eo = pl.pallas_call(
        _grouped_body,
        grid_spec=pltpu.PrefetchScalarGridSpec(
            num_scalar_prefetch=1,
            grid=(nb,),
            in_specs=[
                pl.BlockSpec((BM, dim), lambda i, be: (i, 0)),
                pl.BlockSpec((1, inter, dim), lambda i, be: (be[i], 0, 0)),
                pl.BlockSpec((1, inter, dim), lambda i, be: (be[i], 0, 0)),
                pl.BlockSpec((1, dim, inter), lambda i, be: (be[i], 0, 0)),
                pl.BlockSpec((1, 1, inter), lambda i, be: (be[i], 0, 0)),
                pl.BlockSpec((1, 1, inter), lambda i, be: (be[i], 0, 0)),
                pl.BlockSpec((1, 1, dim), lambda i, be: (be[i], 0, 0)),
            ],
            out_specs=pl.BlockSpec((BM, dim), lambda i, be: (i, 0)),
            scratch_shapes=[
                pltpu.VMEM((inter, dim), jnp.bfloat16),
                pltpu.VMEM((inter, dim), jnp.bfloat16),
                pltpu.VMEM((dim, inter), jnp.bfloat16),
            ],
        ),
        out_shape=jax.ShapeDtypeStruct((npad, dim), jnp.float32),
        compiler_params=pltpu.CompilerParams(
            dimension_semantics=("arbitrary",)),
    )(blk.reshape(-1)[:nb], xs, W1, W3, W2,
      b1.reshape(n_experts, 1, inter), b3.reshape(n_experts, 1, inter),
      b2.reshape(n_experts, 1, dim))

    z = pl.pallas_call(
        _shared_body,
        grid=(t // tb,),
        in_specs=[
            pl.BlockSpec((tb, dim), lambda i: (i, 0)),
            pl.BlockSpec((inter, dim), lambda i: (0, 0)),
            pl.BlockSpec((dim, inter), lambda i: (0, 0)),
            pl.BlockSpec((1, inter), lambda i: (0, 0)),
            pl.BlockSpec((1, dim), lambda i: (0, 0)),
        ],
        out_specs=pl.BlockSpec((tb, dim), lambda i: (i, 0)),
        out_shape=jax.ShapeDtypeStruct((t, dim), jnp.bfloat16),
    )(xf, Ws1, Ws2, bs1.reshape(1, inter), bs2.reshape(1, dim))

    ag, bg = _sc_gather2(eo, pos0, pos1, t)

    y = pl.pallas_call(
        _combine_body,
        grid=(t // tb,),
        in_specs=[
            pl.BlockSpec((tb, dim), lambda i: (i, 0)),
            pl.BlockSpec((tb, dim), lambda i: (i, 0)),
            pl.BlockSpec((tb, dim), lambda i: (i, 0)),
            pl.BlockSpec((tb, 2), lambda i: (i, 0)),
        ],
        out_specs=pl.BlockSpec((tb, dim), lambda i: (i, 0)),
        out_shape=jax.ShapeDtypeStruct((t, dim), jnp.float32),
    )(ag, bg, z, wts)

    return y.reshape(shape)
